# baseline (speedup 1.0000x reference)
import functools

import jax
import jax.numpy as jnp
from jax import lax
from jax.experimental import pallas as pl
from jax.experimental.pallas import tpu as pltpu
from jax.experimental.pallas import tpu_sc as plsc

NUM_SEGS = 128
LANES = 16
CHUNK = 128
_FMIN = float(jnp.finfo(jnp.float32).min)


@functools.cache
def _make_sc_kernel(N, H, S, C):
    info = plsc.get_sparse_core_info()
    NW = info.num_cores * info.num_subcores
    assert S % NW == 0 and H % LANES == 0 and N % LANES == 0
    SPW = S // NW
    F = H // LANES
    CSH = C.bit_length() - 1
    mesh = plsc.VectorSubcoreMesh(core_axis_name="c", subcore_axis_name="s")

    @functools.partial(
        pl.kernel,
        out_type=jax.ShapeDtypeStruct((S * 2 * H,), jnp.float32),
        mesh=mesh,
        scratch_types=[
            pltpu.VMEM((N + LANES,), jnp.int32),
            pltpu.VMEM((C, H), jnp.float32),
            pltpu.VMEM((C, H), jnp.float32),
            pltpu.VMEM((SPW * 2 * H,), jnp.float32),
            pltpu.SemaphoreType.DMA,
            pltpu.SemaphoreType.DMA,
        ],
    )
    def k(x_hbm, batch_hbm, out_hbm, batch_v, buf_a, buf_b, acc_v,
          sem_a, sem_b):
        wid = lax.axis_index("s") * info.num_cores + lax.axis_index("c")
        seg0 = wid * SPW
        pltpu.sync_copy(batch_hbm, batch_v.at[pl.ds(0, N)])

        def bs_body(_, lohis):
            new = []
            for t, (lo, hi) in enumerate(lohis):
                mid = lax.shift_right_logical(lo + hi, 1)
                val = batch_v[pl.ds(mid, LANES)][0]
                pred = val < seg0 + t
                new.append(
                    (jnp.where(pred, mid + 1, lo), jnp.where(pred, hi, mid))
                )
            return tuple(new)

        init = ((jnp.int32(0), jnp.int32(N)),) * (SPW + 1)
        bounds = [lh[0] for lh in lax.fori_loop(0, 17, bs_body, init)]

        zero = jnp.zeros((LANES,), jnp.float32)
        fmin = jnp.full((LANES,), _FMIN, jnp.float32)
        for kseg in range(SPW):
            for f in range(F):
                acc_v[pl.ds(kseg * 2 * H + f * LANES, LANES)] = zero
                acc_v[pl.ds(kseg * 2 * H + H + f * LANES, LANES)] = fmin

        r_begin = bounds[0]
        r_end = bounds[SPW]
        base_a = lax.bitwise_and(r_begin, jnp.int32(~7))
        nfull = lax.shift_right_logical(r_end - base_a, CSH)
        tail_start = base_a + nfull * C
        tail_n = r_end - tail_start

        def start_dma(i, buf, sem):
            start = pl.multiple_of(base_a + i * C, 8)
            pltpu.async_copy(x_hbm.at[pl.ds(start, C)], buf, sem)

        def wait_dma(buf, sem):
            pltpu.make_async_copy(x_hbm.at[pl.ds(0, C)], buf, sem).wait()

        def seg_rows(start, limit, buf):
            for kseg in range(SPW):
                jlo = jnp.maximum(bounds[kseg] - start, 0)
                jhi = jnp.minimum(limit, bounds[kseg + 1] - start)

                @pl.when(jlo < jhi)
                def _(jlo=jlo, jhi=jhi, kseg=kseg):
                    ab = kseg * 2 * H
                    s = [
                        acc_v[pl.ds(ab + f * LANES, LANES)] for f in range(F)
                    ]
                    m = [
                        acc_v[pl.ds(ab + H + f * LANES, LANES)]
                        for f in range(F)
                    ]

                    def row_body(j, car):
                        s, m = car
                        new_s, new_m = [], []
                        for f in range(F):
                            v = buf[j, pl.ds(f * LANES, LANES)]
                            new_s.append(s[f] + v)
                            new_m.append(jnp.maximum(m[f], v))
                        return tuple(new_s), tuple(new_m)

                    s, m = lax.fori_loop(
                        jlo, jhi, row_body, (tuple(s), tuple(m))
                    )
                    for f in range(F):
                        acc_v[pl.ds(ab + f * LANES, LANES)] = s[f]
                        acc_v[pl.ds(ab + H + f * LANES, LANES)] = m[f]

        def compute_acc(i, buf):
            seg_rows(base_a + i * C, C, buf)

        @pl.when(nfull >= 1)
        def _():
            start_dma(0, buf_a, sem_a)

        def pair_body(p, carry):
            i0 = 2 * p
            start_dma(i0 + 1, buf_b, sem_b)
            wait_dma(buf_a, sem_a)
            compute_acc(i0, buf_a)

            @pl.when(i0 + 2 < nfull)
            def _():
                start_dma(i0 + 2, buf_a, sem_a)

            wait_dma(buf_b, sem_b)
            compute_acc(i0 + 1, buf_b)
            return carry

        npairs = lax.shift_right_logical(nfull, 1)
        lax.fori_loop(0, npairs, pair_body, jnp.int32(0))

        @pl.when(lax.bitwise_and(nfull, 1) == 1)
        def _():
            wait_dma(buf_a, sem_a)
            compute_acc(nfull - 1, buf_a)

        G = 8

        for t in range(C // G):

            @pl.when(t * G < tail_n)
            def _(t=t):
                start = pl.multiple_of(tail_start + t * G, G)
                pltpu.async_copy(
                    x_hbm.at[pl.ds(start, G)],
                    buf_a.at[pl.ds(t * G, G)],
                    sem_a,
                )

        for t in range(C // G):

            @pl.when(t * G < tail_n)
            def _(t=t):
                pltpu.make_async_copy(
                    x_hbm.at[pl.ds(0, G)],
                    buf_a.at[pl.ds(t * G, G)],
                    sem_a,
                ).wait()

        seg_rows(tail_start, tail_n, buf_a)

        for kseg in range(SPW):
            ab = kseg * 2 * H
            cnt = bounds[kseg + 1] - bounds[kseg]
            cntf = lax.broadcast_in_dim(cnt, (LANES,), ()).astype(jnp.float32)
            scale = 1.0 / jnp.maximum(cntf, 1.0)
            nonempty = jnp.minimum(cntf, 1.0)
            for f in range(F):
                acc_v[pl.ds(ab + f * LANES, LANES)] = (
                    acc_v[pl.ds(ab + f * LANES, LANES)] * scale
                )
                acc_v[pl.ds(ab + H + f * LANES, LANES)] = (
                    acc_v[pl.ds(ab + H + f * LANES, LANES)] * nonempty
                )
        pltpu.sync_copy(
            acc_v, out_hbm.at[pl.ds(seg0 * 2 * H, SPW * 2 * H)]
        )

    return k


def kernel(x, batch):
    N, H = x.shape
    out = _make_sc_kernel(N, H, NUM_SEGS, CHUNK)(x, batch)
    return out.reshape(NUM_SEGS, 2 * H)
